# baseline (speedup 1.0000x reference)
import jax
import jax.numpy as jnp
from jax import lax
from jax.experimental import pallas as pl
from jax.experimental.pallas import tpu as pltpu
from jax.experimental.pallas import tpu_sc as plsc

VOCAB = 1000000
EMBED_DIM = 64
BATCH = 16384

NUM_CORES = 2
NUM_SUBCORES = 16
LANES = 16
NW = NUM_CORES * NUM_SUBCORES
B_PER_W = BATCH // NW
HALF = B_PER_W // 2
PACK_W = 2 * EMBED_DIM
TCOLS = 4096
TGRID = (VOCAB + TCOLS - 1) // TCOLS
NPACK = TGRID * (TCOLS // 2)


def _pack_table(emb):
    emb_t = emb.T

    def body(x_ref, o_ref):
        x = x_ref[...]
        y = x.T
        o_ref[...] = jnp.concatenate(
            [y[:TCOLS // 2], y[TCOLS // 2:]], axis=1)

    return pl.pallas_call(
        body,
        grid=(TGRID,),
        in_specs=[pl.BlockSpec((EMBED_DIM, TCOLS), lambda i: (0, i))],
        out_specs=pl.BlockSpec((TCOLS // 2, PACK_W), lambda i: (i, 0)),
        out_shape=jax.ShapeDtypeStruct((NPACK, PACK_W), jnp.float32),
    )(emb_t)


def _sc_scores(centers, contexts, packed):
    mesh = plsc.VectorSubcoreMesh(core_axis_name="c", subcore_axis_name="s")

    @jax.jit
    def run(centers, contexts, packed):
        @pl.kernel(
            out_type=jax.ShapeDtypeStruct((BATCH,), jnp.float32),
            mesh=mesh,
            compiler_params=pltpu.CompilerParams(needs_layout_passes=False),
            scratch_types=[
                pltpu.VMEM((B_PER_W,), jnp.int32),
                pltpu.VMEM((B_PER_W,), jnp.int32),
                pltpu.VMEM((4, 128), jnp.int32),
                pltpu.VMEM((4, 128), jnp.int32),
                pltpu.VMEM((HALF, PACK_W), jnp.float32),
                pltpu.VMEM((HALF, PACK_W), jnp.float32),
                pltpu.VMEM((B_PER_W,), jnp.float32),
                pltpu.SemaphoreType.DMA,
            ],
        )
        def k(centers_hbm, contexts_hbm, packed_hbm, out_hbm,
              c_vmem, x_vmem, cq_v, xq_v, u_t, v_t, score_v, sem):
            wid = lax.axis_index("s") * NUM_CORES + lax.axis_index("c")
            base = wid * B_PER_W

            pltpu.sync_copy(centers_hbm.at[pl.ds(base, B_PER_W)], c_vmem)
            pltpu.sync_copy(contexts_hbm.at[pl.ds(base, B_PER_W)], x_vmem)

            def packed_idx(r):
                return (lax.shift_left(
                    lax.shift_right_logical(r, 12), 11) | (r & 2047))

            for j in range(4):
                for t in range(128 // LANES):
                    sl = pl.ds(t * LANES, LANES)
                    cq_v[j, sl] = packed_idx(
                        c_vmem[pl.ds(j * 128 + t * LANES, LANES)])
                    xq_v[j, sl] = packed_idx(
                        x_vmem[pl.ds(j * 128 + t * LANES, LANES)])

            lane = lax.iota(jnp.int32, LANES)

            def half_body(h, _):
                e0 = h * HALF
                j0 = h * 2
                copies = []
                for j in range(2):
                    copies.append(pltpu.async_copy(
                        packed_hbm.at[cq_v.at[j0 + j]],
                        u_t.at[pl.ds(j * 128, 128)], sem))
                    copies.append(pltpu.async_copy(
                        packed_hbm.at[xq_v.at[j0 + j]],
                        v_t.at[pl.ds(j * 128, 128)], sem))
                for cp in copies:
                    cp.wait()

                def grp_body(g, _):
                    rows = g * LANES + lane
                    cpar = lax.shift_right_logical(
                        c_vmem[pl.ds(e0 + g * LANES, LANES)] & 2048, 5)
                    xpar = lax.shift_right_logical(
                        x_vmem[pl.ds(e0 + g * LANES, LANES)] & 2048, 5)
                    acc = jnp.zeros((LANES,), jnp.float32)
                    for c in range(EMBED_DIM):
                        un = plsc.load_gather(u_t, [rows, cpar + c])
                        vn = plsc.load_gather(v_t, [rows, xpar + c])
                        acc = acc + un * vn
                    score_v[pl.ds(e0 + g * LANES, LANES)] = acc
                    return ()

                lax.fori_loop(0, HALF // LANES, grp_body, ())
                return ()

            lax.fori_loop(0, 2, half_body, ())

            pltpu.sync_copy(score_v, out_hbm.at[pl.ds(base, B_PER_W)])

        return k(centers, contexts, packed)

    return run(centers, contexts, packed)


def _tc_loss(scores):
    x2d = scores.reshape(BATCH // 128, 128)

    def body(x_ref, o_ref):
        x = x_ref[...]
        ls = jnp.minimum(x, 0.0) - jnp.log1p(jnp.exp(-jnp.abs(x)))
        o_ref[0, 0] = -jnp.sum(ls)

    out = pl.pallas_call(
        body,
        out_shape=jax.ShapeDtypeStruct((1, 1), jnp.float32),
        out_specs=pl.BlockSpec(memory_space=pltpu.SMEM),
    )(x2d)
    return out.reshape(())


def kernel(centers, contexts, emb):
    packed = _pack_table(emb)
    scores = _sc_scores(centers.astype(jnp.int32), contexts.astype(jnp.int32),
                        packed)
    return _tc_loss(scores)


# speedup vs baseline: 1.3501x; 1.3501x over previous
import jax
import jax.numpy as jnp
from jax import lax
from jax.experimental import pallas as pl
from jax.experimental.pallas import tpu as pltpu
from jax.experimental.pallas import tpu_sc as plsc

VOCAB = 1000000
EMBED_DIM = 64
BATCH = 16384

NUM_CORES = 2
NUM_SUBCORES = 16
LANES = 16
NW = NUM_CORES * NUM_SUBCORES
B_PER_W = BATCH // NW
HALF = B_PER_W // 2
PACK_W = 2 * EMBED_DIM
TCOLS = 16384
TGRID = (VOCAB + TCOLS - 1) // TCOLS
NPACK = TGRID * (TCOLS // 2)


def _pack_table(emb):
    emb_t = emb.T

    def body(x_ref, o_ref):
        x = x_ref[...]
        iot = lax.broadcasted_iota(jnp.int32, (EMBED_DIM, EMBED_DIM), 0)
        eye = jnp.where(
            iot == lax.broadcasted_iota(jnp.int32, (EMBED_DIM, EMBED_DIM), 1),
            1.0, 0.0).astype(jnp.float32)
        y = lax.dot_general(x, eye, (((0,), (0,)), ((), ())),
                            preferred_element_type=jnp.float32)
        o_ref[...] = jnp.concatenate(
            [y[:TCOLS // 2], y[TCOLS // 2:]], axis=1)

    return pl.pallas_call(
        body,
        grid=(TGRID,),
        in_specs=[pl.BlockSpec((EMBED_DIM, TCOLS), lambda i: (0, i))],
        out_specs=pl.BlockSpec((TCOLS // 2, PACK_W), lambda i: (i, 0)),
        out_shape=jax.ShapeDtypeStruct((NPACK, PACK_W), jnp.float32),
    )(emb_t)


def _sc_scores(centers, contexts, packed):
    mesh = plsc.VectorSubcoreMesh(core_axis_name="c", subcore_axis_name="s")

    @jax.jit
    def run(centers, contexts, packed):
        @pl.kernel(
            out_type=jax.ShapeDtypeStruct((BATCH,), jnp.float32),
            mesh=mesh,
            compiler_params=pltpu.CompilerParams(needs_layout_passes=False),
            scratch_types=[
                pltpu.VMEM((B_PER_W,), jnp.int32),
                pltpu.VMEM((B_PER_W,), jnp.int32),
                pltpu.VMEM((4, 128), jnp.int32),
                pltpu.VMEM((4, 128), jnp.int32),
                pltpu.VMEM((HALF, PACK_W), jnp.float32),
                pltpu.VMEM((HALF, PACK_W), jnp.float32),
                pltpu.VMEM((B_PER_W,), jnp.float32),
                pltpu.SemaphoreType.DMA,
            ],
        )
        def k(centers_hbm, contexts_hbm, packed_hbm, out_hbm,
              c_vmem, x_vmem, cq_v, xq_v, u_t, v_t, score_v, sem):
            wid = lax.axis_index("s") * NUM_CORES + lax.axis_index("c")
            base = wid * B_PER_W

            pltpu.sync_copy(centers_hbm.at[pl.ds(base, B_PER_W)], c_vmem)
            pltpu.sync_copy(contexts_hbm.at[pl.ds(base, B_PER_W)], x_vmem)

            def packed_idx(r):
                return (lax.shift_left(
                    lax.shift_right_logical(r, 14), 13) | (r & 8191))

            for j in range(4):
                for t in range(128 // LANES):
                    sl = pl.ds(t * LANES, LANES)
                    cq_v[j, sl] = packed_idx(
                        c_vmem[pl.ds(j * 128 + t * LANES, LANES)])
                    xq_v[j, sl] = packed_idx(
                        x_vmem[pl.ds(j * 128 + t * LANES, LANES)])

            lane = lax.iota(jnp.int32, LANES)

            def half_body(h, _):
                e0 = h * HALF
                j0 = h * 2
                copies = []
                for j in range(2):
                    copies.append(pltpu.async_copy(
                        packed_hbm.at[cq_v.at[j0 + j]],
                        u_t.at[pl.ds(j * 128, 128)], sem))
                    copies.append(pltpu.async_copy(
                        packed_hbm.at[xq_v.at[j0 + j]],
                        v_t.at[pl.ds(j * 128, 128)], sem))
                for cp in copies:
                    cp.wait()

                def grp_body(g, _):
                    rows = g * LANES + lane
                    cpar = lax.shift_right_logical(
                        c_vmem[pl.ds(e0 + g * LANES, LANES)] & 8192, 7)
                    xpar = lax.shift_right_logical(
                        x_vmem[pl.ds(e0 + g * LANES, LANES)] & 8192, 7)
                    acc = jnp.zeros((LANES,), jnp.float32)
                    for c in range(EMBED_DIM):
                        un = plsc.load_gather(u_t, [rows, cpar + c])
                        vn = plsc.load_gather(v_t, [rows, xpar + c])
                        acc = acc + un * vn
                    score_v[pl.ds(e0 + g * LANES, LANES)] = acc
                    return ()

                lax.fori_loop(0, HALF // LANES, grp_body, ())
                return ()

            lax.fori_loop(0, 2, half_body, ())

            pltpu.sync_copy(score_v, out_hbm.at[pl.ds(base, B_PER_W)])

        return k(centers, contexts, packed)

    return run(centers, contexts, packed)


def _tc_loss(scores):
    x2d = scores.reshape(BATCH // 128, 128)

    def body(x_ref, o_ref):
        x = x_ref[...]
        ls = jnp.minimum(x, 0.0) - jnp.log1p(jnp.exp(-jnp.abs(x)))
        o_ref[0, 0] = -jnp.sum(ls)

    out = pl.pallas_call(
        body,
        out_shape=jax.ShapeDtypeStruct((1, 1), jnp.float32),
        out_specs=pl.BlockSpec(memory_space=pltpu.SMEM),
    )(x2d)
    return out.reshape(())


def kernel(centers, contexts, emb):
    packed = _pack_table(emb)
    scores = _sc_scores(centers.astype(jnp.int32), contexts.astype(jnp.int32),
                        packed)
    return _tc_loss(scores)


# speedup vs baseline: 1.4221x; 1.0534x over previous
import jax
import jax.numpy as jnp
from jax import lax
from jax.experimental import pallas as pl
from jax.experimental.pallas import tpu as pltpu
from jax.experimental.pallas import tpu_sc as plsc

VOCAB = 1000000
EMBED_DIM = 64
BATCH = 16384

NUM_CORES = 2
NUM_SUBCORES = 16
LANES = 16
NW = NUM_CORES * NUM_SUBCORES
B_PER_W = BATCH // NW
HALF = B_PER_W // 2
PACK_W = 2 * EMBED_DIM
TCOLS = 32768
TSHIFT = TCOLS.bit_length() - 1
HBIT = TCOLS // 2
TGRID = (VOCAB + TCOLS - 1) // TCOLS
NPACK = TGRID * (TCOLS // 2)


def _pack_table(emb):
    emb_t = emb.T

    def body(x_ref, o_ref):
        x = x_ref[...]
        iot = lax.broadcasted_iota(jnp.int32, (EMBED_DIM, EMBED_DIM), 0)
        eye = jnp.where(
            iot == lax.broadcasted_iota(jnp.int32, (EMBED_DIM, EMBED_DIM), 1),
            1.0, 0.0).astype(jnp.float32)
        y = lax.dot_general(x, eye, (((0,), (0,)), ((), ())),
                            preferred_element_type=jnp.float32)
        o_ref[...] = jnp.concatenate(
            [y[:TCOLS // 2], y[TCOLS // 2:]], axis=1)

    return pl.pallas_call(
        body,
        grid=(TGRID,),
        in_specs=[pl.BlockSpec((EMBED_DIM, TCOLS), lambda i: (0, i))],
        out_specs=pl.BlockSpec((TCOLS // 2, PACK_W), lambda i: (i, 0)),
        out_shape=jax.ShapeDtypeStruct((NPACK, PACK_W), jnp.float32),
    )(emb_t)


def _sc_scores(centers, contexts, packed):
    mesh = plsc.VectorSubcoreMesh(core_axis_name="c", subcore_axis_name="s")

    @jax.jit
    def run(centers, contexts, packed):
        @pl.kernel(
            out_type=jax.ShapeDtypeStruct((BATCH,), jnp.float32),
            mesh=mesh,
            compiler_params=pltpu.CompilerParams(needs_layout_passes=False),
            scratch_types=[
                pltpu.VMEM((B_PER_W,), jnp.int32),
                pltpu.VMEM((B_PER_W,), jnp.int32),
                pltpu.VMEM((4, 128), jnp.int32),
                pltpu.VMEM((4, 128), jnp.int32),
                pltpu.VMEM((HALF, PACK_W), jnp.float32),
                pltpu.VMEM((HALF, PACK_W), jnp.float32),
                pltpu.VMEM((B_PER_W,), jnp.float32),
                pltpu.SemaphoreType.DMA,
            ],
        )
        def k(centers_hbm, contexts_hbm, packed_hbm, out_hbm,
              c_vmem, x_vmem, cq_v, xq_v, u_t, v_t, score_v, sem):
            wid = lax.axis_index("s") * NUM_CORES + lax.axis_index("c")
            base = wid * B_PER_W

            pltpu.sync_copy(centers_hbm.at[pl.ds(base, B_PER_W)], c_vmem)
            pltpu.sync_copy(contexts_hbm.at[pl.ds(base, B_PER_W)], x_vmem)

            def packed_idx(r):
                return (lax.shift_left(
                    lax.shift_right_logical(r, TSHIFT), TSHIFT - 1)
                    | (r & (HBIT - 1)))

            for j in range(4):
                for t in range(128 // LANES):
                    sl = pl.ds(t * LANES, LANES)
                    cq_v[j, sl] = packed_idx(
                        c_vmem[pl.ds(j * 128 + t * LANES, LANES)])
                    xq_v[j, sl] = packed_idx(
                        x_vmem[pl.ds(j * 128 + t * LANES, LANES)])

            lane = lax.iota(jnp.int32, LANES)

            def half_body(h, _):
                e0 = h * HALF
                j0 = h * 2
                copies = []
                for j in range(2):
                    copies.append(pltpu.async_copy(
                        packed_hbm.at[cq_v.at[j0 + j]],
                        u_t.at[pl.ds(j * 128, 128)], sem))
                    copies.append(pltpu.async_copy(
                        packed_hbm.at[xq_v.at[j0 + j]],
                        v_t.at[pl.ds(j * 128, 128)], sem))
                for cp in copies:
                    cp.wait()

                def grp_body(g, _):
                    rows = g * LANES + lane
                    cpar = lax.shift_right_logical(
                        c_vmem[pl.ds(e0 + g * LANES, LANES)] & HBIT,
                        TSHIFT - 7)
                    xpar = lax.shift_right_logical(
                        x_vmem[pl.ds(e0 + g * LANES, LANES)] & HBIT,
                        TSHIFT - 7)
                    acc = jnp.zeros((LANES,), jnp.float32)
                    for c in range(EMBED_DIM):
                        un = plsc.load_gather(u_t, [rows, cpar + c])
                        vn = plsc.load_gather(v_t, [rows, xpar + c])
                        acc = acc + un * vn
                    score_v[pl.ds(e0 + g * LANES, LANES)] = acc
                    return ()

                lax.fori_loop(0, HALF // LANES, grp_body, ())
                return ()

            lax.fori_loop(0, 2, half_body, ())

            pltpu.sync_copy(score_v, out_hbm.at[pl.ds(base, B_PER_W)])

        return k(centers, contexts, packed)

    return run(centers, contexts, packed)


def _tc_loss(scores):
    x2d = scores.reshape(BATCH // 128, 128)

    def body(x_ref, o_ref):
        x = x_ref[...]
        ls = jnp.minimum(x, 0.0) - jnp.log1p(jnp.exp(-jnp.abs(x)))
        o_ref[0, 0] = -jnp.sum(ls)

    out = pl.pallas_call(
        body,
        out_shape=jax.ShapeDtypeStruct((1, 1), jnp.float32),
        out_specs=pl.BlockSpec(memory_space=pltpu.SMEM),
    )(x2d)
    return out.reshape(())


def kernel(centers, contexts, emb):
    packed = _pack_table(emb)
    scores = _sc_scores(centers.astype(jnp.int32), contexts.astype(jnp.int32),
                        packed)
    return _tc_loss(scores)


# speedup vs baseline: 1.4247x; 1.0018x over previous
import jax
import jax.numpy as jnp
from jax import lax
from jax.experimental import pallas as pl
from jax.experimental.pallas import tpu as pltpu
from jax.experimental.pallas import tpu_sc as plsc

VOCAB = 1000000
EMBED_DIM = 64
BATCH = 16384

NUM_CORES = 2
NUM_SUBCORES = 16
LANES = 16
NW = NUM_CORES * NUM_SUBCORES
B_PER_W = BATCH // NW
HALF = B_PER_W // 2
PACK_W = 2 * EMBED_DIM
TCOLS = 32768
TSHIFT = TCOLS.bit_length() - 1
HBIT = TCOLS // 2
TGRID = (VOCAB + TCOLS - 1) // TCOLS
NPACK = TGRID * (TCOLS // 2)


def _pack_table(emb):
    emb_t = emb.T

    def body(x_ref, o_ref):
        x = x_ref[...]
        iot = lax.broadcasted_iota(jnp.int32, (EMBED_DIM, EMBED_DIM), 0)
        eye = jnp.where(
            iot == lax.broadcasted_iota(jnp.int32, (EMBED_DIM, EMBED_DIM), 1),
            1.0, 0.0).astype(jnp.float32)
        y = lax.dot_general(x, eye, (((0,), (0,)), ((), ())),
                            preferred_element_type=jnp.float32)
        o_ref[...] = jnp.concatenate(
            [y[:TCOLS // 2], y[TCOLS // 2:]], axis=1)

    return pl.pallas_call(
        body,
        grid=(TGRID,),
        in_specs=[pl.BlockSpec((EMBED_DIM, TCOLS), lambda i: (0, i))],
        out_specs=pl.BlockSpec((TCOLS // 2, PACK_W), lambda i: (i, 0)),
        out_shape=jax.ShapeDtypeStruct((NPACK, PACK_W), jnp.float32),
    )(emb_t)


def _sc_scores(centers, contexts, packed):
    mesh = plsc.VectorSubcoreMesh(core_axis_name="c", subcore_axis_name="s")

    @jax.jit
    def run(centers, contexts, packed):
        @pl.kernel(
            out_type=jax.ShapeDtypeStruct((BATCH,), jnp.float32),
            mesh=mesh,
            compiler_params=pltpu.CompilerParams(needs_layout_passes=False),
            scratch_types=[
                pltpu.VMEM((B_PER_W,), jnp.int32),
                pltpu.VMEM((B_PER_W,), jnp.int32),
                pltpu.VMEM((4, 128), jnp.int32),
                pltpu.VMEM((4, 128), jnp.int32),
                pltpu.VMEM((HALF, PACK_W), jnp.float32),
                pltpu.VMEM((HALF, PACK_W), jnp.float32),
                pltpu.VMEM((B_PER_W,), jnp.float32),
                pltpu.SemaphoreType.DMA,
                pltpu.SemaphoreType.DMA,
            ],
        )
        def k(centers_hbm, contexts_hbm, packed_hbm, out_hbm,
              c_vmem, x_vmem, cq_v, xq_v, u_t, v_t, score_v, sem0, sem1):
            wid = lax.axis_index("s") * NUM_CORES + lax.axis_index("c")
            base = wid * B_PER_W

            pltpu.sync_copy(centers_hbm.at[pl.ds(base, B_PER_W)], c_vmem)
            pltpu.sync_copy(contexts_hbm.at[pl.ds(base, B_PER_W)], x_vmem)

            def packed_idx(r):
                return (lax.shift_left(
                    lax.shift_right_logical(r, TSHIFT), TSHIFT - 1)
                    | (r & (HBIT - 1)))

            for j in range(4):
                for t in range(128 // LANES):
                    sl = pl.ds(t * LANES, LANES)
                    cq_v[j, sl] = packed_idx(
                        c_vmem[pl.ds(j * 128 + t * LANES, LANES)])
                    xq_v[j, sl] = packed_idx(
                        x_vmem[pl.ds(j * 128 + t * LANES, LANES)])

            lane = lax.iota(jnp.int32, LANES)

            def fire(j):
                slot = j & 1
                sem = sem0 if slot == 0 else sem1
                return [
                    pltpu.async_copy(packed_hbm.at[cq_v.at[j]],
                                     u_t.at[pl.ds(slot * 128, 128)], sem),
                    pltpu.async_copy(packed_hbm.at[xq_v.at[j]],
                                     v_t.at[pl.ds(slot * 128, 128)], sem),
                ]

            def compute(j):
                slot = j & 1
                for g in range(128 // LANES):
                    rows = slot * 128 + g * LANES + lane
                    e0 = j * 128 + g * LANES
                    cpar = lax.shift_right_logical(
                        c_vmem[pl.ds(e0, LANES)] & HBIT, TSHIFT - 7)
                    xpar = lax.shift_right_logical(
                        x_vmem[pl.ds(e0, LANES)] & HBIT, TSHIFT - 7)
                    acc = jnp.zeros((LANES,), jnp.float32)
                    for c in range(EMBED_DIM):
                        un = plsc.load_gather(u_t, [rows, cpar + c])
                        vn = plsc.load_gather(v_t, [rows, xpar + c])
                        acc = acc + un * vn
                    score_v[pl.ds(e0, LANES)] = acc

            pending = fire(0)
            for j in range(4):
                nxt = fire(j + 1) if j + 1 < 4 else []
                for cp in pending:
                    cp.wait()
                compute(j)
                pending = nxt

            pltpu.sync_copy(score_v, out_hbm.at[pl.ds(base, B_PER_W)])

        return k(centers, contexts, packed)

    return run(centers, contexts, packed)


def _tc_loss(scores):
    x2d = scores.reshape(BATCH // 128, 128)

    def body(x_ref, o_ref):
        x = x_ref[...]
        ls = jnp.minimum(x, 0.0) - jnp.log1p(jnp.exp(-jnp.abs(x)))
        o_ref[0, 0] = -jnp.sum(ls)

    out = pl.pallas_call(
        body,
        out_shape=jax.ShapeDtypeStruct((1, 1), jnp.float32),
        out_specs=pl.BlockSpec(memory_space=pltpu.SMEM),
    )(x2d)
    return out.reshape(())


def kernel(centers, contexts, emb):
    packed = _pack_table(emb)
    scores = _sc_scores(centers.astype(jnp.int32), contexts.astype(jnp.int32),
                        packed)
    return _tc_loss(scores)
